# baseline (speedup 1.0000x reference)
import functools

import jax
import jax.numpy as jnp
from jax import lax
from jax.experimental import pallas as pl
from jax.experimental.pallas import tpu as pltpu
from jax.experimental.pallas import tpu_sc as plsc

_N = 10000
_NHID = 512
_E = 160000
_ROWS = 1000
_NSLAB = 8
_FS = 64
_EPAD = 163840
_NB = 1280
_NBT = _NB // 16
_NPAD = 10240
_STRIPE = _NPAD // 16


def _mm_bias_act_body(a_ref, b_ref, bias_ref, o_ref, *, act):
    acc = jnp.dot(a_ref[...], b_ref[...], preferred_element_type=jnp.float32)
    acc = acc + bias_ref[...]
    if act == "relu":
        acc = jnp.maximum(acc, 0.0)
    o_ref[...] = acc.astype(o_ref.dtype)


def _mm_bias_act(a, b, bias, act=None, rows=_ROWS, out_dtype=jnp.float32):
    n, k = a.shape
    m = b.shape[1]
    return pl.pallas_call(
        functools.partial(_mm_bias_act_body, act=act),
        grid=(n // rows,),
        in_specs=[
            pl.BlockSpec((rows, k), lambda i: (i, 0)),
            pl.BlockSpec((k, m), lambda i: (0, 0)),
            pl.BlockSpec((1, m), lambda i: (0, 0)),
        ],
        out_specs=pl.BlockSpec((rows, m), lambda i: (i, 0)),
        out_shape=jax.ShapeDtypeStruct((n, m), out_dtype),
    )(a, b, bias.reshape(1, -1))


def _mm_body(a_ref, b_ref, o_ref):
    o_ref[...] = jnp.dot(a_ref[...], b_ref[...],
                         preferred_element_type=jnp.float32)


def _mm_small(a, b):
    n, k = a.shape
    m = b.shape[1]
    return pl.pallas_call(
        _mm_body,
        in_specs=[pl.BlockSpec((n, k), lambda: (0, 0)),
                  pl.BlockSpec((k, m), lambda: (0, 0))],
        out_specs=pl.BlockSpec((n, m), lambda: (0, 0)),
        out_shape=jax.ShapeDtypeStruct((n, m), jnp.float32),
    )(a, b)


def _hws_resid_body(x_ref, w_ref, w2_ref, dinv_ref, rb_ref, o_ref, r_ref):
    acc = jnp.dot(x_ref[...], w_ref[...], preferred_element_type=jnp.float32)
    acc = acc * dinv_ref[...]
    o_ref[0] = acc[:, :_FS]
    o_ref[1] = acc[:, _FS:]
    r_ref[...] = jnp.dot(x_ref[...], w2_ref[...],
                         preferred_element_type=jnp.float32) + rb_ref[...]


def _hws_resid(x, conv_w_t, w2_t, dinv, rbias, rows=_ROWS):
    n, k = x.shape
    return pl.pallas_call(
        _hws_resid_body,
        grid=(n // rows, _NSLAB // 2),
        in_specs=[
            pl.BlockSpec((rows, k), lambda i, s: (i, 0)),
            pl.BlockSpec((k, 128), lambda i, s: (0, s)),
            pl.BlockSpec((k, 128), lambda i, s: (0, s)),
            pl.BlockSpec((rows, 1), lambda i, s: (i, 0)),
            pl.BlockSpec((1, 128), lambda i, s: (0, s)),
        ],
        out_specs=[
            pl.BlockSpec((2, rows, _FS), lambda i, s: (s, i, 0)),
            pl.BlockSpec((rows, 128), lambda i, s: (i, s)),
        ],
        out_shape=[
            jax.ShapeDtypeStruct((_NSLAB, n, _FS), jnp.float32),
            jax.ShapeDtypeStruct((n, _NHID), jnp.float32),
        ],
    )(x, conv_w_t, w2_t, dinv.reshape(-1, 1), rbias.reshape(1, -1))


def _epi_body(s_ref, hws_ref, r_ref, dinv_ref, o_ref):
    s128 = jnp.concatenate([s_ref[0], s_ref[1]], axis=1)
    hws128 = jnp.concatenate([hws_ref[0], hws_ref[1]], axis=1)
    g = dinv_ref[...] * (s128 + hws128) - r_ref[...]
    o_ref[...] = jnp.maximum(g, 0.0).astype(o_ref.dtype)


def _layer_epilogue(s_t, hws_t, r, dinv, rows=_ROWS):
    n = r.shape[0]
    return pl.pallas_call(
        _epi_body,
        grid=(n // rows, _NSLAB // 2),
        in_specs=[
            pl.BlockSpec((2, rows, _FS), lambda i, s: (s, i, 0)),
            pl.BlockSpec((2, rows, _FS), lambda i, s: (s, i, 0)),
            pl.BlockSpec((rows, 128), lambda i, s: (i, s)),
            pl.BlockSpec((rows, 1), lambda i, s: (i, 0)),
        ],
        out_specs=pl.BlockSpec((rows, 128), lambda i, s: (i, s)),
        out_shape=jax.ShapeDtypeStruct((n, _NHID), jnp.bfloat16),
    )(s_t, hws_t, r, dinv.reshape(-1, 1))


def _sc_msgpass(hws_t, src2d, dst2d, zeros128):
    mesh = plsc.VectorSubcoreMesh(core_axis_name="c", subcore_axis_name="s")

    @functools.partial(
        pl.kernel,
        mesh=mesh,
        compiler_params=pltpu.CompilerParams(use_tc_tiling_on_sc=False),
        out_type=jax.ShapeDtypeStruct((_NSLAB, _NPAD, _FS), jnp.float32),
        scratch_types=[
            pltpu.VMEM((_NBT, 128), jnp.int32),
            pltpu.VMEM((_NBT, 128), jnp.int32),
            pltpu.VMEM((128, _FS), jnp.float32),
            pltpu.VMEM((128, _FS), jnp.float32),
            pltpu.VMEM((128, _FS), jnp.float32),
            pltpu.VMEM((128, _FS), jnp.float32),
            pltpu.VMEM((128, _FS), jnp.float32),
            pltpu.VMEM_SHARED((_NPAD, _FS), jnp.float32),
            pltpu.SemaphoreType.DMA,
            pltpu.SemaphoreType.DMA,
            pltpu.SemaphoreType.DMA,
            pltpu.SemaphoreType.DMA,
            pltpu.SemaphoreType.DMA,
            pltpu.SemaphoreType.DMA,
            pltpu.SemaphoreType.DMA,
            pltpu.SemaphoreType.DMA,
        ],
    )
    def k(hws_hbm, src_hbm, dst_hbm, zero_hbm, out_hbm,
          src_v, dst_v, buf0, buf1, buf2, buf3, zero_v, accum,
          gsem0, gsem1, gsem2, gsem3, ssem0, ssem1, ssem2, ssem3):
        c = lax.axis_index("c")
        s = lax.axis_index("s")
        pltpu.sync_copy(src_hbm.at[pl.ds(s * _NBT, _NBT)], src_v)
        pltpu.sync_copy(dst_hbm.at[pl.ds(s * _NBT, _NBT)], dst_v)
        pltpu.sync_copy(zero_hbm, zero_v)
        bufs = (buf0, buf1, buf2, buf3)
        gsems = (gsem0, gsem1, gsem2, gsem3)
        ssems = (ssem0, ssem1, ssem2, ssem3)
        for p in range(_NSLAB // 2):
            slab = c * (_NSLAB // 2) + p
            table = hws_hbm.at[slab]
            for z in range(_STRIPE // 128):
                pltpu.sync_copy(zero_v,
                                accum.at[pl.ds(s * _STRIPE + z * 128, 128)])
            plsc.subcore_barrier()
            pltpu.make_async_copy(table.at[src_v.at[0]], buf0, gsem0).start()
            pltpu.make_async_copy(table.at[src_v.at[1]], buf1, gsem1).start()

            def body(g, carry):
                for b in range(4):
                    j = g * 4 + b
                    bn = (b + 2) % 4
                    pltpu.make_async_copy(table.at[src_v.at[j]],
                                          bufs[b], gsems[b]).wait()
                    pltpu.async_copy(bufs[b], accum.at[dst_v.at[j]],
                                     ssems[b], add=True)
                    jn = j + 2

                    @pl.when(jn < _NBT)
                    def _():
                        @pl.when(j >= 2)
                        def _():
                            pltpu.make_async_copy(
                                bufs[bn], accum.at[dst_v.at[j]],
                                ssems[bn]).wait()

                        pltpu.make_async_copy(table.at[src_v.at[jn]],
                                              bufs[bn], gsems[bn]).start()

                return carry

            lax.fori_loop(0, _NBT // 4, body, 0)
            for b in range(4):
                pltpu.make_async_copy(bufs[b], accum.at[dst_v.at[_NBT - 1]],
                                      ssems[b]).wait()
            plsc.subcore_barrier()
            pltpu.sync_copy(accum.at[pl.ds(s * _STRIPE, _STRIPE)],
                            out_hbm.at[slab, pl.ds(s * _STRIPE, _STRIPE)])
            plsc.subcore_barrier()

    return k(hws_t, src2d, dst2d, zeros128)


def kernel(x, edge_index, enc_W, enc_b, conv_W, conv_b, res_W, res_b, dec_W, dec_b):
    src = edge_index[0]
    dst = edge_index[1]

    deg = jnp.ones((_N,), jnp.float32).at[dst].add(1.0)
    dinv = lax.rsqrt(deg)

    pad = _EPAD - _E
    src2d = jnp.concatenate(
        [src, jnp.zeros((pad,), jnp.int32)]).reshape(_NB, 128)
    dst2d = jnp.concatenate(
        [dst, jnp.full((pad,), _N, jnp.int32)]).reshape(_NB, 128)
    zeros128 = jnp.zeros((128, _FS), jnp.float32)

    w2_t = _mm_small(conv_W.T, res_W.T).astype(jnp.bfloat16)
    conv_w_t = conv_W.T.astype(jnp.bfloat16)

    X = _mm_bias_act(x.astype(jnp.bfloat16), enc_W.T.astype(jnp.bfloat16),
                     enc_b, act="relu", out_dtype=jnp.bfloat16)

    rbias = res_b - conv_b
    for _ in range(4):
        hws_t, r = _hws_resid(X, conv_w_t, w2_t, dinv, rbias)
        s_t = _sc_msgpass(hws_t, src2d, dst2d, zeros128)
        X = _layer_epilogue(s_t, hws_t, r, dinv)

    return _mm_bias_act(X, dec_W.T.astype(jnp.bfloat16), dec_b, act=None)


# speedup vs baseline: 1.0379x; 1.0379x over previous
import functools

import jax
import jax.numpy as jnp
from jax import lax
from jax.experimental import pallas as pl
from jax.experimental.pallas import tpu as pltpu
from jax.experimental.pallas import tpu_sc as plsc

_N = 10000
_NHID = 512
_E = 160000
_ROWS = 1000
_NSLAB = 8
_FS = 64
_EPAD = 163840
_NB = 1280
_NBT = _NB // 16
_NPT = _NBT // 2
_NPAD = 10240
_STRIPE = _NPAD // 16


def _mm_bias_act_body(a_ref, b_ref, bias_ref, o_ref, *, act):
    acc = jnp.dot(a_ref[...], b_ref[...], preferred_element_type=jnp.float32)
    acc = acc + bias_ref[...]
    if act == "relu":
        acc = jnp.maximum(acc, 0.0)
    o_ref[...] = acc.astype(o_ref.dtype)


def _mm_bias_act(a, b, bias, act=None, rows=_ROWS, out_dtype=jnp.float32):
    n, k = a.shape
    m = b.shape[1]
    return pl.pallas_call(
        functools.partial(_mm_bias_act_body, act=act),
        grid=(n // rows,),
        in_specs=[
            pl.BlockSpec((rows, k), lambda i: (i, 0)),
            pl.BlockSpec((k, m), lambda i: (0, 0)),
            pl.BlockSpec((1, m), lambda i: (0, 0)),
        ],
        out_specs=pl.BlockSpec((rows, m), lambda i: (i, 0)),
        out_shape=jax.ShapeDtypeStruct((n, m), out_dtype),
    )(a, b, bias.reshape(1, -1))


def _mm_body(a_ref, b_ref, o_ref):
    o_ref[...] = jnp.dot(a_ref[...], b_ref[...],
                         preferred_element_type=jnp.float32)


def _mm_small(a, b):
    n, k = a.shape
    m = b.shape[1]
    return pl.pallas_call(
        _mm_body,
        in_specs=[pl.BlockSpec((n, k), lambda: (0, 0)),
                  pl.BlockSpec((k, m), lambda: (0, 0))],
        out_specs=pl.BlockSpec((n, m), lambda: (0, 0)),
        out_shape=jax.ShapeDtypeStruct((n, m), jnp.float32),
    )(a, b)


def _hws_resid_body(x_ref, w_ref, w2_ref, dinv_ref, rb_ref, o_ref, r_ref):
    acc = jnp.dot(x_ref[...], w_ref[...], preferred_element_type=jnp.float32)
    acc = acc * dinv_ref[...]
    o_ref[0] = acc[:, :_FS]
    o_ref[1] = acc[:, _FS:]
    r_ref[...] = jnp.dot(x_ref[...], w2_ref[...],
                         preferred_element_type=jnp.float32) + rb_ref[...]


def _hws_resid(x, conv_w_t, w2_t, dinv, rbias, rows=_ROWS):
    n, k = x.shape
    return pl.pallas_call(
        _hws_resid_body,
        grid=(n // rows, _NSLAB // 2),
        in_specs=[
            pl.BlockSpec((rows, k), lambda i, s: (i, 0)),
            pl.BlockSpec((k, 128), lambda i, s: (0, s)),
            pl.BlockSpec((k, 128), lambda i, s: (0, s)),
            pl.BlockSpec((rows, 1), lambda i, s: (i, 0)),
            pl.BlockSpec((1, 128), lambda i, s: (0, s)),
        ],
        out_specs=[
            pl.BlockSpec((2, rows, _FS), lambda i, s: (s, i, 0)),
            pl.BlockSpec((rows, 128), lambda i, s: (i, s)),
        ],
        out_shape=[
            jax.ShapeDtypeStruct((_NSLAB, n, _FS), jnp.float32),
            jax.ShapeDtypeStruct((n, _NHID), jnp.float32),
        ],
    )(x, conv_w_t, w2_t, dinv.reshape(-1, 1), rbias.reshape(1, -1))


def _epi_body(s_ref, hws_ref, r_ref, dinv_ref, o_ref):
    s128 = jnp.concatenate([s_ref[0], s_ref[1]], axis=1)
    hws128 = jnp.concatenate([hws_ref[0], hws_ref[1]], axis=1)
    g = dinv_ref[...] * (s128 + hws128) - r_ref[...]
    o_ref[...] = jnp.maximum(g, 0.0).astype(o_ref.dtype)


def _layer_epilogue(s_t, hws_t, r, dinv, rows=_ROWS):
    n = r.shape[0]
    return pl.pallas_call(
        _epi_body,
        grid=(n // rows, _NSLAB // 2),
        in_specs=[
            pl.BlockSpec((2, rows, _FS), lambda i, s: (s, i, 0)),
            pl.BlockSpec((2, rows, _FS), lambda i, s: (s, i, 0)),
            pl.BlockSpec((rows, 128), lambda i, s: (i, s)),
            pl.BlockSpec((rows, 1), lambda i, s: (i, 0)),
        ],
        out_specs=pl.BlockSpec((rows, 128), lambda i, s: (i, s)),
        out_shape=jax.ShapeDtypeStruct((n, _NHID), jnp.float32),
    )(s_t, hws_t, r, dinv.reshape(-1, 1))


def _sc_msgpass(hws_t, src2d, dst2d, zeros128):
    mesh = plsc.VectorSubcoreMesh(core_axis_name="c", subcore_axis_name="s")

    @functools.partial(
        pl.kernel,
        mesh=mesh,
        compiler_params=pltpu.CompilerParams(use_tc_tiling_on_sc=False),
        out_type=jax.ShapeDtypeStruct((_NSLAB, _NPAD, _FS), jnp.float32),
        scratch_types=[
            pltpu.VMEM((_NBT, 128), jnp.int32),
            pltpu.VMEM((_NBT, 128), jnp.int32),
            pltpu.VMEM((128, _FS), jnp.float32),
            pltpu.VMEM((128, _FS), jnp.float32),
            pltpu.VMEM((128, _FS), jnp.float32),
            pltpu.VMEM((128, _FS), jnp.float32),
            pltpu.VMEM((128, _FS), jnp.float32),
            pltpu.VMEM_SHARED((_NPAD, _FS), jnp.float32),
            pltpu.SemaphoreType.DMA,
            pltpu.SemaphoreType.DMA,
            pltpu.SemaphoreType.DMA,
            pltpu.SemaphoreType.DMA,
            pltpu.SemaphoreType.DMA,
            pltpu.SemaphoreType.DMA,
            pltpu.SemaphoreType.DMA,
            pltpu.SemaphoreType.DMA,
        ],
    )
    def k(hws_hbm, src_hbm, dst_hbm, zero_hbm, out_hbm,
          src_v, dst_v, buf0, buf1, buf2, buf3, zero_v, accum,
          gsem0, gsem1, gsem2, gsem3, ssem0, ssem1, ssem2, ssem3):
        c = lax.axis_index("c")
        s = lax.axis_index("s")
        pltpu.sync_copy(src_hbm.at[pl.ds(s * _NBT, _NBT)], src_v)
        pltpu.sync_copy(dst_hbm.at[pl.ds(s * _NBT, _NBT)], dst_v)
        pltpu.sync_copy(zero_hbm, zero_v)
        bufs = (buf0, buf1, buf2, buf3)
        gsems = (gsem0, gsem1, gsem2, gsem3)
        ssems = (ssem0, ssem1, ssem2, ssem3)
        for p in range(_NSLAB // 2):
            slab = c * (_NSLAB // 2) + p
            table = hws_hbm.at[slab]
            for z in range(_STRIPE // 128):
                pltpu.sync_copy(zero_v,
                                accum.at[pl.ds(s * _STRIPE + z * 128, 128)])
            plsc.subcore_barrier()
            pltpu.make_async_copy(table.at[src_v.at[0]], buf0, gsem0).start()
            pltpu.make_async_copy(table.at[src_v.at[1]], buf1, gsem1).start()

            def body(g, carry):
                for b in range(4):
                    j = g * 4 + b
                    bn = (b + 2) % 4
                    pltpu.make_async_copy(table.at[src_v.at[j]],
                                          bufs[b], gsems[b]).wait()
                    pltpu.async_copy(bufs[b], accum.at[dst_v.at[j]],
                                     ssems[b], add=True)
                    jn = j + 2

                    @pl.when(jn < _NBT)
                    def _():
                        @pl.when(j >= 2)
                        def _():
                            pltpu.make_async_copy(
                                bufs[bn], accum.at[dst_v.at[j]],
                                ssems[bn]).wait()

                        pltpu.make_async_copy(table.at[src_v.at[jn]],
                                              bufs[bn], gsems[bn]).start()

                return carry

            lax.fori_loop(0, _NBT // 4, body, 0)
            for b in range(4):
                pltpu.make_async_copy(bufs[b], accum.at[dst_v.at[_NBT - 1]],
                                      ssems[b]).wait()
            plsc.subcore_barrier()
            pltpu.sync_copy(accum.at[pl.ds(s * _STRIPE, _STRIPE)],
                            out_hbm.at[slab, pl.ds(s * _STRIPE, _STRIPE)])
            plsc.subcore_barrier()

    return k(hws_t, src2d, dst2d, zeros128)


def kernel(x, edge_index, enc_W, enc_b, conv_W, conv_b, res_W, res_b, dec_W, dec_b):
    src = edge_index[0]
    dst = edge_index[1]

    deg = jnp.ones((_N,), jnp.float32).at[dst].add(1.0)
    dinv = lax.rsqrt(deg)

    pad = _EPAD - _E
    src2d = jnp.concatenate(
        [src, jnp.zeros((pad,), jnp.int32)]).reshape(_NB, 128)
    dst2d = jnp.concatenate(
        [dst, jnp.full((pad,), _N, jnp.int32)]).reshape(_NB, 128)
    zeros128 = jnp.zeros((128, _FS), jnp.float32)

    w2_t = _mm_small(conv_W.T, res_W.T)
    conv_w_t = conv_W.T

    X = _mm_bias_act(x, enc_W.T, enc_b, act="relu")

    rbias = res_b - conv_b
    for _ in range(4):
        hws_t, r = _hws_resid(X, conv_w_t, w2_t, dinv, rbias)
        s_t = _sc_msgpass(hws_t, src2d, dst2d, zeros128)
        X = _layer_epilogue(s_t, hws_t, r, dinv)

    return _mm_bias_act(X, dec_W.T, dec_b, act=None)
